# baseline (speedup 1.0000x reference)
import jax
import jax.numpy as jnp
from jax import lax
from jax.experimental import pallas as pl
from jax.experimental.pallas import tpu as pltpu

_MARGIN = 0.5
_N = 4096
_ROWS = 256
_COLS = 1024
_NI = _N // _ROWS
_NJ = _N // _COLS
_BIG = 1e30


def _pair_kernel(p2_ref, t2_ref, pr_ref, tr_ref, out_ref, negv_ref, pc_ref, acc_ref):
    trow = tr_ref[...]
    negv_ref[...] = jnp.where(
        trow == 0, pr_ref[...] + jnp.float32(_MARGIN), jnp.float32(-_BIG)
    )
    posm = jnp.where(t2_ref[...] == 1, p2_ref[...], jnp.float32(_BIG))
    chunk_iota = lax.broadcasted_iota(jnp.int32, (_NI, 1), 0)

    acc_ref[...] = jnp.zeros_like(acc_ref)

    def body_i(i, _):
        onehot = (chunk_iota == i).astype(jnp.float32)
        pc_ref[...] = lax.dot_general(
            posm,
            onehot,
            (((0,), (0,)), ((), ())),
            preferred_element_type=jnp.float32,
        )

        def body_j(j, _):
            neg8 = jnp.broadcast_to(
                negv_ref[:, pl.ds(j * _COLS, _COLS)], (8, _COLS)
            )
            acc = jnp.maximum(neg8 - pc_ref[0:8, :], jnp.float32(0.0))
            for s in range(1, _ROWS // 8):
                pb = pc_ref[s * 8:(s + 1) * 8, :]
                acc = acc + jnp.maximum(neg8 - pb, jnp.float32(0.0))
            acc_ref[...] += acc
            return 0

        return lax.fori_loop(0, _NJ, body_j, 0)

    lax.fori_loop(0, _NI, body_i, 0)

    total = jnp.sum(acc_ref[...])
    npos = jnp.sum((trow == 1).astype(jnp.float32))
    nneg = jnp.sum((trow == 0).astype(jnp.float32))
    denom = npos * nneg
    out_ref[0, 0] = jnp.where(
        denom > 0, total / jnp.maximum(denom, jnp.float32(1.0)), jnp.float32(0.0)
    )


def kernel(y_pred, y_true):
    out = pl.pallas_call(
        _pair_kernel,
        in_specs=[
            pl.BlockSpec((_NI, _ROWS), lambda: (0, 0)),
            pl.BlockSpec((_NI, _ROWS), lambda: (0, 0)),
            pl.BlockSpec((1, _N), lambda: (0, 0)),
            pl.BlockSpec((1, _N), lambda: (0, 0)),
        ],
        out_specs=pl.BlockSpec(memory_space=pltpu.SMEM),
        out_shape=jax.ShapeDtypeStruct((1, 1), jnp.float32),
        scratch_shapes=[
            pltpu.VMEM((1, _N), jnp.float32),
            pltpu.VMEM((_ROWS, 1), jnp.float32),
            pltpu.VMEM((8, _COLS), jnp.float32),
        ],
    )(
        y_pred.reshape(_NI, _ROWS),
        y_true.reshape(_NI, _ROWS),
        y_pred.reshape(1, _N),
        y_true.reshape(1, _N),
    )
    return out[0, 0]


# speedup vs baseline: 1.1372x; 1.1372x over previous
import jax
import jax.numpy as jnp
from jax import lax
from jax.experimental import pallas as pl
from jax.experimental.pallas import tpu as pltpu

_MARGIN = 0.5
_N = 4096
_ROWS = 256
_COLS = 1024
_NI = _N // _ROWS
_NJ = _N // _COLS
_BIG = 1e30


def _pair_kernel(p2_ref, t2_ref, pr_ref, tr_ref, out_ref, negv_ref, pc_ref, acc_ref):
    trow = tr_ref[...]
    negv_ref[...] = jnp.where(
        trow == 0, pr_ref[...] + jnp.float32(_MARGIN), jnp.float32(-_BIG)
    )
    posm = jnp.where(t2_ref[...] == 1, p2_ref[...], jnp.float32(_BIG))
    chunk_iota = lax.broadcasted_iota(jnp.int32, (_NI, 1), 0)

    acc_ref[...] = jnp.zeros_like(acc_ref)

    def body_i(i, _):
        onehot = (chunk_iota == i).astype(jnp.float32)
        pc_ref[...] = lax.dot_general(
            posm,
            onehot,
            (((0,), (0,)), ((), ())),
            preferred_element_type=jnp.float32,
        )

        def body_j(j, _):
            neg8 = jnp.broadcast_to(
                negv_ref[:, pl.ds(j * _COLS, _COLS)], (8, _COLS)
            )
            acc = jnp.maximum(neg8 - pc_ref[0:8, :], jnp.float32(0.0))
            for s in range(1, _ROWS // 8):
                pb = pc_ref[s * 8:(s + 1) * 8, :]
                acc = acc + jnp.maximum(neg8 - pb, jnp.float32(0.0))
            acc_ref[...] += acc
            return 0

        return lax.fori_loop(0, _NJ, body_j, 0, unroll=2)

    lax.fori_loop(0, _NI, body_i, 0)

    total = jnp.sum(acc_ref[...])
    npos = jnp.sum((trow == 1).astype(jnp.float32))
    nneg = jnp.sum((trow == 0).astype(jnp.float32))
    denom = npos * nneg
    out_ref[0, 0] = jnp.where(
        denom > 0, total / jnp.maximum(denom, jnp.float32(1.0)), jnp.float32(0.0)
    )


def kernel(y_pred, y_true):
    out = pl.pallas_call(
        _pair_kernel,
        in_specs=[
            pl.BlockSpec((_NI, _ROWS), lambda: (0, 0)),
            pl.BlockSpec((_NI, _ROWS), lambda: (0, 0)),
            pl.BlockSpec((1, _N), lambda: (0, 0)),
            pl.BlockSpec((1, _N), lambda: (0, 0)),
        ],
        out_specs=pl.BlockSpec(memory_space=pltpu.SMEM),
        out_shape=jax.ShapeDtypeStruct((1, 1), jnp.float32),
        scratch_shapes=[
            pltpu.VMEM((1, _N), jnp.float32),
            pltpu.VMEM((_ROWS, 1), jnp.float32),
            pltpu.VMEM((8, _COLS), jnp.float32),
        ],
    )(
        y_pred.reshape(_NI, _ROWS),
        y_true.reshape(_NI, _ROWS),
        y_pred.reshape(1, _N),
        y_true.reshape(1, _N),
    )
    return out[0, 0]


# speedup vs baseline: 1.2030x; 1.0578x over previous
import jax
import jax.numpy as jnp
from jax import lax
from jax.experimental import pallas as pl
from jax.experimental.pallas import tpu as pltpu

_MARGIN = 0.5
_N = 4096
_ROWS = 256
_COLS = 1024
_NI = _N // _ROWS
_NJ = _N // _COLS
_BIG = 1e30


def _pair_kernel(p2_ref, t2_ref, pr_ref, tr_ref, out_ref, negv_ref):
    trow = tr_ref[...]
    negv_ref[...] = jnp.where(
        trow == 0, pr_ref[...] + jnp.float32(_MARGIN), jnp.float32(-_BIG)
    )
    posm = jnp.where(t2_ref[...] == 1, p2_ref[...], jnp.float32(_BIG))
    chunk_iota = lax.broadcasted_iota(jnp.int32, (_NI, 1), 0)

    def body_i(i, acc):
        onehot = (chunk_iota == i).astype(jnp.float32)
        pos_chunk = lax.dot_general(
            posm,
            onehot,
            (((0,), (0,)), ((), ())),
            preferred_element_type=jnp.float32,
        )

        def body_j(j, acc):
            neg_chunk = negv_ref[:, pl.ds(j * _COLS, _COLS)]
            for s in range(_ROWS // 8):
                pb = lax.slice(pos_chunk, (s * 8, 0), (s * 8 + 8, 1))
                acc = acc + jnp.maximum(neg_chunk - pb, jnp.float32(0.0))
            return acc

        return lax.fori_loop(0, _NJ, body_j, acc, unroll=True)

    acc = lax.fori_loop(
        0, _NI, body_i, jnp.zeros((8, _COLS), jnp.float32)
    )

    total = jnp.sum(acc)
    npos = jnp.sum((trow == 1).astype(jnp.float32))
    nneg = jnp.sum((trow == 0).astype(jnp.float32))
    denom = npos * nneg
    out_ref[0, 0] = jnp.where(
        denom > 0, total / jnp.maximum(denom, jnp.float32(1.0)), jnp.float32(0.0)
    )


def kernel(y_pred, y_true):
    out = pl.pallas_call(
        _pair_kernel,
        in_specs=[
            pl.BlockSpec((_NI, _ROWS), lambda: (0, 0)),
            pl.BlockSpec((_NI, _ROWS), lambda: (0, 0)),
            pl.BlockSpec((1, _N), lambda: (0, 0)),
            pl.BlockSpec((1, _N), lambda: (0, 0)),
        ],
        out_specs=pl.BlockSpec(memory_space=pltpu.SMEM),
        out_shape=jax.ShapeDtypeStruct((1, 1), jnp.float32),
        scratch_shapes=[
            pltpu.VMEM((1, _N), jnp.float32),
        ],
    )(
        y_pred.reshape(_NI, _ROWS),
        y_true.reshape(_NI, _ROWS),
        y_pred.reshape(1, _N),
        y_true.reshape(1, _N),
    )
    return out[0, 0]
